# baseline (speedup 1.0000x reference)
import jax
import jax.numpy as jnp
from jax import lax
from jax.experimental import pallas as pl
from jax.experimental.pallas import tpu as pltpu
from jax.experimental.pallas import tpu_sc as plsc

B = 16384
F = 26
FH = 13
D = 13
VOCAB = 100000
NC = 2
NS = 16
L = 16
NW = NC * NS
R = B // NW
RV = R // L


def _worker_base():
    wid = lax.axis_index("s") * NC + lax.axis_index("c")
    return wid * R


def _stage_idx(xs_hbm, idx_v, base, f0, sem):
    return [
        pltpu.make_async_copy(
            xs_hbm.at[pl.ds((f0 + f) * B + base, R)],
            idx_v.at[pl.ds(f * R, R)],
            sem,
        )
        for f in range(FH)
    ]


def _add_offsets(idx_v):
    def add_off(j, _):
        sl = pl.ds(j * L, L)
        idx_v[sl] = idx_v[sl] + (j // RV) * VOCAB
        return _

    lax.fori_loop(0, FH * RV, add_off, 0, unroll=4)


def _body_a(xs_hbm, tab_hbm, out_hbm, idx_v, g_v, acc_v, sem):
    base = _worker_base()
    stage = _stage_idx(xs_hbm, idx_v, base, 0, sem)
    for c in stage:
        c.start()
    for c in stage:
        c.wait()
    _add_offsets(idx_v)
    gather = pltpu.make_async_copy(tab_hbm.at[idx_v], g_v, sem)
    gather.start()
    gather.wait()

    def reduce_one(j, _):
        sl = pl.ds(j * L, L)
        acc = g_v[sl]
        for f in range(1, FH):
            acc = acc + g_v[pl.ds(f * R + j * L, L)]
        acc_v[sl] = acc
        return _

    lax.fori_loop(0, RV, reduce_one, 0, unroll=2)
    pltpu.sync_copy(acc_v, out_hbm.at[pl.ds(base, R)])


def _body_b(xs_hbm, xd_hbm, w_hbm, tab_hbm, part_hbm, out_hbm,
            idx_v, g_v, xd_v, w_v, part_v, acc_v, sem):
    base = _worker_base()
    stage = _stage_idx(xs_hbm, idx_v, base, FH, sem)
    stage += [
        pltpu.make_async_copy(
            xd_hbm.at[pl.ds(d * B + base, R)], xd_v.at[pl.ds(d * R, R)], sem
        )
        for d in range(D)
    ]
    stage.append(pltpu.make_async_copy(w_hbm, w_v, sem))
    stage.append(pltpu.make_async_copy(part_hbm.at[pl.ds(base, R)], part_v, sem))
    for c in stage:
        c.start()
    for c in stage:
        c.wait()
    _add_offsets(idx_v)
    gather = pltpu.make_async_copy(tab_hbm.at[idx_v], g_v, sem)
    gather.start()
    gather.wait()

    wrows = [w_v[d] for d in range(D)]

    def reduce_one(j, _):
        sl = pl.ds(j * L, L)
        acc = part_v[sl] + g_v[sl]
        for f in range(1, FH):
            acc = acc + g_v[pl.ds(f * R + j * L, L)]
        for d in range(D):
            acc = acc + xd_v[pl.ds(d * R + j * L, L)] * wrows[d]
        acc_v[sl] = acc
        return _

    lax.fori_loop(0, RV, reduce_one, 0, unroll=2)
    pltpu.sync_copy(acc_v, out_hbm.at[pl.ds(base, R)])


@jax.jit
def _linear_sc(xs_flat, xd_flat, w_rep, tab_a, tab_b):
    mesh = plsc.VectorSubcoreMesh(core_axis_name="c", subcore_axis_name="s")
    run_a = pl.kernel(
        _body_a,
        out_type=jax.ShapeDtypeStruct((B,), jnp.float32),
        mesh=mesh,
        scratch_types=[
            pltpu.VMEM((FH * R,), jnp.int32),
            pltpu.VMEM((FH * R,), jnp.float32),
            pltpu.VMEM((R,), jnp.float32),
            pltpu.SemaphoreType.DMA,
        ],
    )
    partial = run_a(xs_flat, tab_a)
    run_b = pl.kernel(
        _body_b,
        out_type=jax.ShapeDtypeStruct((B,), jnp.float32),
        mesh=mesh,
        scratch_types=[
            pltpu.VMEM((FH * R,), jnp.int32),
            pltpu.VMEM((FH * R,), jnp.float32),
            pltpu.VMEM((D * R,), jnp.float32),
            pltpu.VMEM((D, L), jnp.float32),
            pltpu.VMEM((R,), jnp.float32),
            pltpu.VMEM((R,), jnp.float32),
            pltpu.SemaphoreType.DMA,
        ],
    )
    return run_b(xs_flat, xd_flat, w_rep, tab_b, partial)


def kernel(X_sparse, X_dense, tables, W_dense):
    xs_flat = X_sparse.astype(jnp.int32).T.reshape(F * B)
    xd_flat = X_dense.T.reshape(D * B)
    w_rep = jnp.broadcast_to(W_dense.reshape(D, 1), (D, L))
    tab2d = lax.optimization_barrier(tables[:, :, 0])
    tab_a = tab2d[:FH].reshape(FH * VOCAB)
    tab_b = tab2d[FH:].reshape(FH * VOCAB)
    out = _linear_sc(xs_flat, xd_flat, w_rep, tab_a, tab_b)
    return out.reshape(B, 1)


# speedup vs baseline: 1.0306x; 1.0306x over previous
import jax
import jax.numpy as jnp
from jax import lax
from jax.experimental import pallas as pl
from jax.experimental.pallas import tpu as pltpu
from jax.experimental.pallas import tpu_sc as plsc

B = 16384
F = 26
FA = 16
FB = F - FA
D = 13
VOCAB = 100000
NC = 2
NS = 16
L = 16
NW = NC * NS
R = B // NW
RV = R // L


def _worker_base():
    wid = lax.axis_index("s") * NC + lax.axis_index("c")
    return wid * R


def _stage_idx(xs_hbm, idx_v, base, f0, nf, sem):
    return [
        pltpu.make_async_copy(
            xs_hbm.at[pl.ds((f0 + f) * B + base, R)],
            idx_v.at[pl.ds(f * R, R)],
            sem,
        )
        for f in range(nf)
    ]


def _add_offsets(idx_v, nf):
    def add_off(j, _):
        sl = pl.ds(j * L, L)
        idx_v[sl] = idx_v[sl] + (j // RV) * VOCAB
        return _

    lax.fori_loop(0, nf * RV, add_off, 0, unroll=4)


def _body_a(xs_hbm, tab_hbm, out_hbm, idx_v, g_v, acc_v, sem):
    base = _worker_base()
    stage = _stage_idx(xs_hbm, idx_v, base, 0, FA, sem)
    for c in stage:
        c.start()
    for c in stage:
        c.wait()
    _add_offsets(idx_v, FA)
    gather = pltpu.make_async_copy(tab_hbm.at[idx_v], g_v, sem)
    gather.start()
    gather.wait()

    def reduce_one(j, _):
        sl = pl.ds(j * L, L)
        acc = g_v[sl]
        for f in range(1, FA):
            acc = acc + g_v[pl.ds(f * R + j * L, L)]
        acc_v[sl] = acc
        return _

    lax.fori_loop(0, RV, reduce_one, 0, unroll=2)
    pltpu.sync_copy(acc_v, out_hbm.at[pl.ds(base, R)])


def _body_b(xs_hbm, xd_hbm, w_hbm, tab_hbm, part_hbm, out_hbm,
            idx_v, g_v, xd_v, w_v, part_v, acc_v, sem):
    base = _worker_base()
    stage = _stage_idx(xs_hbm, idx_v, base, FA, FB, sem)
    stage += [
        pltpu.make_async_copy(
            xd_hbm.at[pl.ds(d * B + base, R)], xd_v.at[pl.ds(d * R, R)], sem
        )
        for d in range(D)
    ]
    stage.append(pltpu.make_async_copy(w_hbm, w_v, sem))
    stage.append(pltpu.make_async_copy(part_hbm.at[pl.ds(base, R)], part_v, sem))
    for c in stage:
        c.start()
    for c in stage:
        c.wait()
    _add_offsets(idx_v, FB)
    gather = pltpu.make_async_copy(tab_hbm.at[idx_v], g_v, sem)
    gather.start()
    gather.wait()

    wrows = [w_v[d] for d in range(D)]

    def reduce_one(j, _):
        sl = pl.ds(j * L, L)
        acc = part_v[sl] + g_v[sl]
        for f in range(1, FB):
            acc = acc + g_v[pl.ds(f * R + j * L, L)]
        for d in range(D):
            acc = acc + xd_v[pl.ds(d * R + j * L, L)] * wrows[d]
        acc_v[sl] = acc
        return _

    lax.fori_loop(0, RV, reduce_one, 0, unroll=2)
    pltpu.sync_copy(acc_v, out_hbm.at[pl.ds(base, R)])


@jax.jit
def _linear_sc(xs_flat, xd_flat, w_rep, tab_a, tab_b):
    mesh = plsc.VectorSubcoreMesh(core_axis_name="c", subcore_axis_name="s")
    run_a = pl.kernel(
        _body_a,
        out_type=jax.ShapeDtypeStruct((B,), jnp.float32),
        mesh=mesh,
        scratch_types=[
            pltpu.VMEM((FA * R,), jnp.int32),
            pltpu.VMEM((FA * R,), jnp.float32),
            pltpu.VMEM((R,), jnp.float32),
            pltpu.SemaphoreType.DMA,
        ],
    )
    partial = run_a(xs_flat, tab_a)
    run_b = pl.kernel(
        _body_b,
        out_type=jax.ShapeDtypeStruct((B,), jnp.float32),
        mesh=mesh,
        scratch_types=[
            pltpu.VMEM((FB * R,), jnp.int32),
            pltpu.VMEM((FB * R,), jnp.float32),
            pltpu.VMEM((D * R,), jnp.float32),
            pltpu.VMEM((D, L), jnp.float32),
            pltpu.VMEM((R,), jnp.float32),
            pltpu.VMEM((R,), jnp.float32),
            pltpu.SemaphoreType.DMA,
        ],
    )
    return run_b(xs_flat, xd_flat, w_rep, tab_b, partial)


def kernel(X_sparse, X_dense, tables, W_dense):
    xs_flat = X_sparse.astype(jnp.int32).T.reshape(F * B)
    xd_flat = X_dense.T.reshape(D * B)
    w_rep = jnp.broadcast_to(W_dense.reshape(D, 1), (D, L))
    tab2d = lax.optimization_barrier(tables[:, :, 0])
    tab_a = tab2d[:FA].reshape(FA * VOCAB)
    tab_b = tab2d[FA:].reshape(FB * VOCAB)
    out = _linear_sc(xs_flat, xd_flat, w_rep, tab_a, tab_b)
    return out.reshape(B, 1)


# speedup vs baseline: 1.1534x; 1.1192x over previous
import functools

import jax
import jax.numpy as jnp
from jax import lax
from jax.experimental import pallas as pl
from jax.experimental.pallas import tpu as pltpu
from jax.experimental.pallas import tpu_sc as plsc

B = 16384
F = 26
D = 13
VOCAB = 100000
VOCAB_P = 100096
NC = 2
NS = 16
L = 16
NW = NC * NS
R = B // NW
RV = R // L


def _sc_body(xs_hbm, xd_hbm, w_hbm, tab_hbm, out_hbm,
             idx_v, g_v, xd_v, w_v, acc_v, sem):
    cid = lax.axis_index("c")
    sid = lax.axis_index("s")
    wid = sid * NC + cid
    base = wid * R

    stage = [
        pltpu.make_async_copy(
            xs_hbm.at[pl.ds(f * B + base, R)], idx_v.at[pl.ds(f * R, R)], sem
        )
        for f in range(F)
    ] + [
        pltpu.make_async_copy(
            xd_hbm.at[pl.ds(d * B + base, R)], xd_v.at[pl.ds(d * R, R)], sem
        )
        for d in range(D)
    ] + [pltpu.make_async_copy(w_hbm, w_v, sem)]
    for c in stage:
        c.start()
    for c in stage:
        c.wait()

    def add_off(j, _):
        sl = pl.ds(j * L, L)
        off = (j // RV) * VOCAB
        idx_v[sl] = idx_v[sl] + off
        return _

    lax.fori_loop(0, F * RV, add_off, 0, unroll=4)

    gather = pltpu.make_async_copy(tab_hbm.at[idx_v], g_v, sem)
    gather.start()
    gather.wait()

    wrows = [w_v[d] for d in range(D)]

    def reduce_one(j, _):
        sl = pl.ds(j * L, L)
        acc = g_v[sl]
        for f in range(1, F):
            acc = acc + g_v[pl.ds(f * R + j * L, L)]
        for d in range(D):
            acc = acc + xd_v[pl.ds(d * R + j * L, L)] * wrows[d]
        acc_v[sl] = acc
        return _

    lax.fori_loop(0, RV, reduce_one, 0, unroll=2)

    pltpu.sync_copy(acc_v, out_hbm.at[pl.ds(base, R)])


@jax.jit
def _linear_sc(xs_blocks, xd_blocks, w_rep, tab_flat):
    mesh = plsc.VectorSubcoreMesh(core_axis_name="c", subcore_axis_name="s")
    run = pl.kernel(
        _sc_body,
        out_type=jax.ShapeDtypeStruct((B,), jnp.float32),
        mesh=mesh,
        scratch_types=[
            pltpu.VMEM((F * R,), jnp.int32),
            pltpu.VMEM((F * R,), jnp.float32),
            pltpu.VMEM((D * R,), jnp.float32),
            pltpu.VMEM((D, L), jnp.float32),
            pltpu.VMEM((R,), jnp.float32),
            pltpu.SemaphoreType.DMA,
        ],
    )
    return run(xs_blocks, xd_blocks, w_rep, tab_flat)


def kernel(X_sparse, X_dense, tables, W_dense):
    xs_blocks = X_sparse.astype(jnp.int32).T.reshape(F * B)
    xd_blocks = X_dense.T.reshape(D * B)
    w_rep = jnp.broadcast_to(W_dense.reshape(D, 1), (D, L))
    tab2d = lax.optimization_barrier(tables[:, :, 0])
    tab_pad = tab2d.reshape(F * VOCAB)
    out = _linear_sc(xs_blocks, xd_blocks, w_rep, tab_pad)
    return out.reshape(B, 1)
